# baseline (speedup 1.0000x reference)
import functools

import jax
import jax.numpy as jnp
import numpy as np
from jax import lax
from jax.experimental import pallas as pl
from jax.experimental.pallas import tpu as pltpu
from jax.experimental.pallas import tpu_sc as plsc

H = 128
MAX_NB = 15
N_ATOMS = 25000
N_BONDS = 50000
N_MESS = 15000
N_TOT = N_MESS + N_BONDS
N_MOLS = 625
APM = 40
AF = 35
BF = 40

NC, NS = 2, 16
NW = NC * NS
CHUNK = 8


NBUF = 2
SLAB = CHUNK * MAX_NB

@functools.cache
def _make_gather_sum(n_rows_pad):
    per_w = n_rows_pad // NW
    n_chunks = per_w // CHUNK
    assert per_w * NW == n_rows_pad and n_chunks * CHUNK == per_w
    assert n_chunks % 2 == 0
    mesh = plsc.VectorSubcoreMesh(core_axis_name="c", subcore_axis_name="s",
                                  num_cores=NC, num_subcores=NS)

    @functools.partial(
        pl.kernel,
        mesh=mesh,
        out_type=jax.ShapeDtypeStruct((n_rows_pad, H), jnp.float32),
        scratch_types=[
            pltpu.VMEM((n_chunks, SLAB), jnp.int32),
            pltpu.VMEM((SLAB, H), jnp.float32),
            pltpu.VMEM((SLAB, H), jnp.float32),
            pltpu.VMEM((CHUNK, H), jnp.float32),
            pltpu.VMEM((CHUNK, H), jnp.float32),
            pltpu.SemaphoreType.DMA,
            pltpu.SemaphoreType.DMA,
            pltpu.SemaphoreType.DMA,
            pltpu.SemaphoreType.DMA,
        ],
    )
    def gsum(table_hbm, idx_hbm, out_hbm, idx_all, rows0, rows1, acc0, acc1,
             sem0, sem1, st0, st1):
        wid = lax.axis_index("s") * NC + lax.axis_index("c")
        base = wid * per_w
        pltpu.sync_copy(idx_hbm.at[wid], idx_all)

        def fire(ci, rows_v, sem):
            pltpu.async_copy(table_hbm.at[idx_all.at[ci]], rows_v, sem)

        def reduce_store(ci, rows_v, sem, acc_v, st):
            pltpu.make_async_copy(table_hbm.at[idx_all.at[ci]], rows_v,
                                  sem).wait()

            @pl.when(ci >= 2)
            def _():
                pltpu.make_async_copy(
                    acc_v, out_hbm.at[pl.ds(base, CHUNK)], st).wait()

            for b in range(CHUNK):
                for v in range(H // 16):
                    sl = pl.ds(v * 16, 16)
                    acc = rows_v[b * MAX_NB, sl]
                    for j in range(1, MAX_NB):
                        acc = acc + rows_v[b * MAX_NB + j, sl]
                    acc_v[b, sl] = acc
            pltpu.async_copy(acc_v, out_hbm.at[pl.ds(base + ci * CHUNK, CHUNK)],
                             st)

        fire(0, rows0, sem0)

        def body(i, carry):
            ci = i * 2
            fire(ci + 1, rows1, sem1)
            reduce_store(ci, rows0, sem0, acc0, st0)

            @pl.when(ci + 2 < n_chunks)
            def _():
                fire(ci + 2, rows0, sem0)

            reduce_store(ci + 1, rows1, sem1, acc1, st1)
            return carry

        lax.fori_loop(0, n_chunks // 2, body, 0)
        pltpu.make_async_copy(acc0, out_hbm.at[pl.ds(base, CHUNK)], st0).wait()
        pltpu.make_async_copy(acc1, out_hbm.at[pl.ds(base, CHUNK)], st1).wait()

    return gsum


def _gather_bonds(table, idx):
    return _make_gather_sum(50176)(table, idx)


def _gather_atoms(table, idx):
    return _make_gather_sum(25600)(table, idx)


def _prep_idx(g, n_pad):
    g = jnp.pad(g, ((0, n_pad - g.shape[0]), (0, 0)))
    return g.reshape(NW, n_pad // NW // CHUNK, CHUNK * MAX_NB)


_MM_DIMS = (((1,), (1,)), ((), ()))


def _tc1_body(fb_ref, wi_ref, tree_ref, bin_ref, msg_ref):
    i = pl.program_id(0)
    b = lax.dot_general(fb_ref[...], wi_ref[...], _MM_DIMS,
                        preferred_element_type=jnp.float32)
    bin_ref[...] = b
    msg_ref[...] = jnp.where(i < 15, tree_ref[...], jnp.maximum(b, 0.0))


def _tc2_body(tree_ref, bin_ref, nei_ref, wh_ref, msg_ref):
    i = pl.program_id(0)
    nm = lax.dot_general(nei_ref[...], wh_ref[...], _MM_DIMS,
                         preferred_element_type=jnp.float32)
    msg_ref[...] = jnp.where(i < 15, tree_ref[...],
                             jnp.maximum(bin_ref[...] + nm, 0.0))


def _tc3_body(fat_ref, nei_ref, wo1_ref, wo2_ref, par_ref, hid_ref, st_ref):
    i = pl.program_id(0)
    a = lax.dot_general(fat_ref[...], wo1_ref[...], _MM_DIMS,
                        preferred_element_type=jnp.float32)
    a = a + lax.dot_general(nei_ref[...], wo2_ref[...], _MM_DIMS,
                            preferred_element_type=jnp.float32)
    a = a + par_ref[0:1, :]
    hid_ref[...] = a

    @pl.when(i == 0)
    def _():
        st_ref[...] = jnp.zeros_like(st_ref)

    st_ref[0:1, :] += jnp.sum(a, axis=0, keepdims=True)
    st_ref[1:2, :] += jnp.sum(a * a, axis=0, keepdims=True)


def _tc4_body(hid_ref, st_ref, par_ref, mol_ref):
    inv_n = 1.0 / N_ATOMS
    mean = st_ref[0:1, :] * inv_n
    var = st_ref[1:2, :] * inv_n - mean * mean
    inv = lax.rsqrt(var + 1e-5)
    scale = inv * par_ref[1:2, :]
    shift = par_ref[2:3, :] - mean * scale
    x = hid_ref[...]
    y = jnp.maximum(x * scale[:, None, :] + shift[:, None, :], 0.0)
    mol_ref[...] = jnp.sum(y, axis=1) * (1.0 / APM)


def kernel(fatoms, fbonds, agraph, bgraph, scope, tree_message, W_i, W_h,
           W_o, b_o, bn_gamma, bn_beta):
    del scope
    bg = _prep_idx(bgraph, 50176)
    ag = _prep_idx(agraph, 25600)
    params = (jnp.zeros((8, H), jnp.float32)
              .at[0].set(b_o).at[1].set(bn_gamma).at[2].set(bn_beta))
    wo1 = W_o[:, :AF]
    wo2 = W_o[:, AF:]

    bspec = pl.BlockSpec((1000, H), lambda i: (jnp.maximum(i - 15, 0), 0))
    tspec = pl.BlockSpec((1000, H), lambda i: (jnp.minimum(i, 14), 0))

    binput, msg = pl.pallas_call(
        _tc1_body,
        grid=(65,),
        in_specs=[
            pl.BlockSpec((1000, BF), lambda i: (jnp.maximum(i - 15, 0), 0)),
            pl.BlockSpec((H, BF), lambda i: (0, 0)),
            tspec,
        ],
        out_specs=[bspec, pl.BlockSpec((1000, H), lambda i: (i, 0))],
        out_shape=[jax.ShapeDtypeStruct((N_BONDS, H), jnp.float32),
                   jax.ShapeDtypeStruct((N_TOT, H), jnp.float32)],
    )(fbonds, W_i, tree_message)

    for _ in range(2):
        nei = _gather_bonds(msg, bg)[:N_BONDS]
        msg = pl.pallas_call(
            _tc2_body,
            grid=(65,),
            in_specs=[tspec, bspec, bspec,
                      pl.BlockSpec((H, H), lambda i: (0, 0))],
            out_specs=pl.BlockSpec((1000, H), lambda i: (i, 0)),
            out_shape=jax.ShapeDtypeStruct((N_TOT, H), jnp.float32),
        )(tree_message, binput, nei, W_h)

    nei_a = _gather_atoms(msg, ag)[:N_ATOMS]

    hid, stats = pl.pallas_call(
        _tc3_body,
        grid=(25,),
        in_specs=[
            pl.BlockSpec((1000, AF), lambda i: (i, 0)),
            pl.BlockSpec((1000, H), lambda i: (i, 0)),
            pl.BlockSpec((H, AF), lambda i: (0, 0)),
            pl.BlockSpec((H, H), lambda i: (0, 0)),
            pl.BlockSpec((8, H), lambda i: (0, 0)),
        ],
        out_specs=[pl.BlockSpec((1000, H), lambda i: (i, 0)),
                   pl.BlockSpec((8, H), lambda i: (0, 0))],
        out_shape=[jax.ShapeDtypeStruct((N_ATOMS, H), jnp.float32),
                   jax.ShapeDtypeStruct((8, H), jnp.float32)],
    )(fatoms, nei_a, wo1, wo2, params)

    mol_vecs = pl.pallas_call(
        _tc4_body,
        grid=(1,),
        in_specs=[
            pl.BlockSpec((N_MOLS, APM, H), lambda i: (0, 0, 0)),
            pl.BlockSpec((8, H), lambda i: (0, 0)),
            pl.BlockSpec((8, H), lambda i: (0, 0)),
        ],
        out_specs=pl.BlockSpec((N_MOLS, H), lambda i: (0, 0)),
        out_shape=jax.ShapeDtypeStruct((N_MOLS, H), jnp.float32),
    )(hid.reshape(N_MOLS, APM, H), stats, params)

    return mol_vecs


# speedup vs baseline: 1.1791x; 1.1791x over previous
import functools

import jax
import jax.numpy as jnp
import numpy as np
from jax import lax
from jax.experimental import pallas as pl
from jax.experimental.pallas import tpu as pltpu
from jax.experimental.pallas import tpu_sc as plsc

H = 128
MAX_NB = 15
N_ATOMS = 25000
N_BONDS = 50000
N_MESS = 15000
N_TOT = N_MESS + N_BONDS
N_MOLS = 625
APM = 40
AF = 35
BF = 40

NC, NS = 2, 16
NW = NC * NS
CHUNK = 8


NBUF = 2
SLAB = CHUNK * MAX_NB

@functools.cache
def _make_gather_sum(n_rows_pad):
    per_w = n_rows_pad // NW
    n_chunks = per_w // CHUNK
    assert per_w * NW == n_rows_pad and n_chunks * CHUNK == per_w
    assert n_chunks % 2 == 0
    mesh = plsc.VectorSubcoreMesh(core_axis_name="c", subcore_axis_name="s",
                                  num_cores=NC, num_subcores=NS)

    @functools.partial(
        pl.kernel,
        mesh=mesh,
        out_type=jax.ShapeDtypeStruct((n_rows_pad, H), jnp.float32),
        scratch_types=[
            pltpu.VMEM((n_chunks, SLAB), jnp.int32),
            pltpu.VMEM((SLAB, H), jnp.float32),
            pltpu.VMEM((SLAB, H), jnp.float32),
            pltpu.VMEM((CHUNK, H), jnp.float32),
            pltpu.VMEM((CHUNK, H), jnp.float32),
            pltpu.SemaphoreType.DMA,
            pltpu.SemaphoreType.DMA,
            pltpu.SemaphoreType.DMA,
            pltpu.SemaphoreType.DMA,
        ],
    )
    def gsum(table_hbm, idx_hbm, out_hbm, idx_all, rows0, rows1, acc0, acc1,
             sem0, sem1, st0, st1):
        wid = lax.axis_index("s") * NC + lax.axis_index("c")
        base = wid * per_w
        pltpu.sync_copy(idx_hbm.at[wid], idx_all)

        def fire(ci, rows_v, sem):
            pltpu.async_copy(table_hbm.at[idx_all.at[ci]], rows_v, sem)

        def reduce_store(ci, rows_v, sem, acc_v, st):
            pltpu.make_async_copy(table_hbm.at[idx_all.at[ci]], rows_v,
                                  sem).wait()

            @pl.when(ci >= 2)
            def _():
                pltpu.make_async_copy(
                    acc_v, out_hbm.at[pl.ds(base, CHUNK)], st).wait()

            for b in range(CHUNK):
                for v in range(H // 16):
                    sl = pl.ds(v * 16, 16)
                    acc = rows_v[b * MAX_NB, sl]
                    for j in range(1, MAX_NB):
                        acc = acc + rows_v[b * MAX_NB + j, sl]
                    acc_v[b, sl] = acc
            pltpu.async_copy(acc_v, out_hbm.at[pl.ds(base + ci * CHUNK, CHUNK)],
                             st)

        fire(0, rows0, sem0)

        def body(i, carry):
            ci = i * 2
            fire(ci + 1, rows1, sem1)
            reduce_store(ci, rows0, sem0, acc0, st0)

            @pl.when(ci + 2 < n_chunks)
            def _():
                fire(ci + 2, rows0, sem0)

            reduce_store(ci + 1, rows1, sem1, acc1, st1)
            return carry

        lax.fori_loop(0, n_chunks // 2, body, 0)
        pltpu.make_async_copy(acc0, out_hbm.at[pl.ds(base, CHUNK)], st0).wait()
        pltpu.make_async_copy(acc1, out_hbm.at[pl.ds(base, CHUNK)], st1).wait()

    return gsum


def _gather_bonds(table, idx):
    return _make_gather_sum(50176)(table, idx)


def _gather_atoms(table, idx):
    return _make_gather_sum(25088)(table, idx)


def _prep_idx(g, n_pad):
    g = jnp.pad(g, ((0, n_pad - g.shape[0]), (0, 0)))
    return g.reshape(NW, n_pad // NW // CHUNK, CHUNK * MAX_NB)


_MM_DIMS = (((1,), (1,)), ((), ()))


def _tc1_body(fb_ref, wi_ref, tree_ref, bin_ref, msg_ref):
    i = pl.program_id(0)
    b = lax.dot_general(fb_ref[...], wi_ref[...], _MM_DIMS,
                        preferred_element_type=jnp.float32)
    bin_ref[...] = b
    msg_ref[...] = jnp.where(i < 15, tree_ref[...], jnp.maximum(b, 0.0))


def _tc2_body(tree_ref, bin_ref, nei_ref, wh_ref, msg_ref):
    i = pl.program_id(0)
    nm = lax.dot_general(nei_ref[...], wh_ref[...], _MM_DIMS,
                         preferred_element_type=jnp.float32)
    msg_ref[...] = jnp.where(i < 15, tree_ref[...],
                             jnp.maximum(bin_ref[...] + nm, 0.0))


def _tc3_body(fat_ref, nei_ref, wo1_ref, wo2_ref, par_ref, hid_ref, st_ref):
    i = pl.program_id(0)
    a = lax.dot_general(fat_ref[...], wo1_ref[...], _MM_DIMS,
                        preferred_element_type=jnp.float32)
    a = a + lax.dot_general(nei_ref[...], wo2_ref[...], _MM_DIMS,
                            preferred_element_type=jnp.float32)
    a = a + par_ref[0:1, :]
    hid_ref[...] = a

    @pl.when(i == 0)
    def _():
        st_ref[...] = jnp.zeros_like(st_ref)

    st_ref[0:1, :] += jnp.sum(a, axis=0, keepdims=True)
    st_ref[1:2, :] += jnp.sum(a * a, axis=0, keepdims=True)


def _tc4_body(hid_ref, st_ref, par_ref, mol_ref):
    inv_n = 1.0 / N_ATOMS
    mean = st_ref[0:1, :] * inv_n
    var = st_ref[1:2, :] * inv_n - mean * mean
    inv = lax.rsqrt(var + 1e-5)
    scale = inv * par_ref[1:2, :]
    shift = par_ref[2:3, :] - mean * scale
    x = hid_ref[...]
    y = jnp.maximum(x * scale[:, None, :] + shift[:, None, :], 0.0)
    mol_ref[...] = jnp.sum(y, axis=1) * (1.0 / APM)


def kernel(fatoms, fbonds, agraph, bgraph, scope, tree_message, W_i, W_h,
           W_o, b_o, bn_gamma, bn_beta):
    del scope
    bg = _prep_idx(bgraph, 50176)
    ag = _prep_idx(agraph, 25088)
    params = (jnp.zeros((8, H), jnp.float32)
              .at[0].set(b_o).at[1].set(bn_gamma).at[2].set(bn_beta))
    wo1 = W_o[:, :AF]
    wo2 = W_o[:, AF:]

    bspec = pl.BlockSpec((1000, H), lambda i: (jnp.maximum(i - 15, 0), 0))
    tspec = pl.BlockSpec((1000, H), lambda i: (jnp.minimum(i, 14), 0))

    binput, msg = pl.pallas_call(
        _tc1_body,
        grid=(65,),
        in_specs=[
            pl.BlockSpec((1000, BF), lambda i: (jnp.maximum(i - 15, 0), 0)),
            pl.BlockSpec((H, BF), lambda i: (0, 0)),
            tspec,
        ],
        out_specs=[bspec, pl.BlockSpec((1000, H), lambda i: (i, 0))],
        out_shape=[jax.ShapeDtypeStruct((N_BONDS, H), jnp.float32),
                   jax.ShapeDtypeStruct((N_TOT, H), jnp.float32)],
    )(fbonds, W_i, tree_message)

    for _ in range(2):
        nei = _gather_bonds(msg, bg)[:N_BONDS]
        msg = pl.pallas_call(
            _tc2_body,
            grid=(65,),
            in_specs=[tspec, bspec, bspec,
                      pl.BlockSpec((H, H), lambda i: (0, 0))],
            out_specs=pl.BlockSpec((1000, H), lambda i: (i, 0)),
            out_shape=jax.ShapeDtypeStruct((N_TOT, H), jnp.float32),
        )(tree_message, binput, nei, W_h)

    nei_a = _gather_atoms(msg, ag)[:N_ATOMS]

    hid, stats = pl.pallas_call(
        _tc3_body,
        grid=(25,),
        in_specs=[
            pl.BlockSpec((1000, AF), lambda i: (i, 0)),
            pl.BlockSpec((1000, H), lambda i: (i, 0)),
            pl.BlockSpec((H, AF), lambda i: (0, 0)),
            pl.BlockSpec((H, H), lambda i: (0, 0)),
            pl.BlockSpec((8, H), lambda i: (0, 0)),
        ],
        out_specs=[pl.BlockSpec((1000, H), lambda i: (i, 0)),
                   pl.BlockSpec((8, H), lambda i: (0, 0))],
        out_shape=[jax.ShapeDtypeStruct((N_ATOMS, H), jnp.float32),
                   jax.ShapeDtypeStruct((8, H), jnp.float32)],
    )(fatoms, nei_a, wo1, wo2, params)

    mol_vecs = pl.pallas_call(
        _tc4_body,
        grid=(1,),
        in_specs=[
            pl.BlockSpec((N_MOLS, APM, H), lambda i: (0, 0, 0)),
            pl.BlockSpec((8, H), lambda i: (0, 0)),
            pl.BlockSpec((8, H), lambda i: (0, 0)),
        ],
        out_specs=pl.BlockSpec((N_MOLS, H), lambda i: (0, 0)),
        out_shape=jax.ShapeDtypeStruct((N_MOLS, H), jnp.float32),
    )(hid.reshape(N_MOLS, APM, H), stats, params)

    return mol_vecs


# speedup vs baseline: 1.2606x; 1.0691x over previous
import functools

import jax
import jax.numpy as jnp
import numpy as np
from jax import lax
from jax.experimental import pallas as pl
from jax.experimental.pallas import tpu as pltpu
from jax.experimental.pallas import tpu_sc as plsc

H = 128
MAX_NB = 15
N_ATOMS = 25000
N_BONDS = 50000
N_MESS = 15000
N_TOT = N_MESS + N_BONDS
N_MOLS = 625
APM = 40
AF = 35
BF = 40

NC, NS = 2, 16
NW = NC * NS
CHUNK = 8


NBUF = 2
SLAB = CHUNK * MAX_NB

@functools.cache
def _make_gather_sum(n_rows_pad):
    per_w = n_rows_pad // NW
    n_chunks = per_w // CHUNK
    assert per_w * NW == n_rows_pad and n_chunks * CHUNK == per_w
    assert n_chunks % 2 == 0
    mesh = plsc.VectorSubcoreMesh(core_axis_name="c", subcore_axis_name="s",
                                  num_cores=NC, num_subcores=NS)

    @functools.partial(
        pl.kernel,
        mesh=mesh,
        out_type=jax.ShapeDtypeStruct((n_rows_pad, H), jnp.float32),
        scratch_types=[
            pltpu.VMEM((n_chunks, SLAB), jnp.int32),
            pltpu.VMEM((SLAB, H), jnp.float32),
            pltpu.VMEM((SLAB, H), jnp.float32),
            pltpu.VMEM((CHUNK, H), jnp.float32),
            pltpu.VMEM((CHUNK, H), jnp.float32),
            pltpu.SemaphoreType.DMA,
            pltpu.SemaphoreType.DMA,
            pltpu.SemaphoreType.DMA,
            pltpu.SemaphoreType.DMA,
        ],
    )
    def gsum(table_hbm, idx_hbm, out_hbm, idx_all, rows0, rows1, acc0, acc1,
             sem0, sem1, st0, st1):
        wid = lax.axis_index("s") * NC + lax.axis_index("c")
        base = wid * per_w
        pltpu.sync_copy(idx_hbm.at[wid], idx_all)

        def fire(ci, rows_v, sem):
            pltpu.async_copy(table_hbm.at[idx_all.at[ci]], rows_v, sem)

        def reduce_store(ci, rows_v, sem, acc_v, st):
            pltpu.make_async_copy(table_hbm.at[idx_all.at[ci]], rows_v,
                                  sem).wait()

            @pl.when(ci >= 2)
            def _():
                pltpu.make_async_copy(
                    acc_v, out_hbm.at[pl.ds(base, CHUNK)], st).wait()

            for b in range(CHUNK):
                for v in range(H // 16):
                    sl = pl.ds(v * 16, 16)
                    acc = rows_v[b * MAX_NB, sl]
                    for j in range(1, MAX_NB):
                        acc = acc + rows_v[b * MAX_NB + j, sl]
                    acc_v[b, sl] = acc
            pltpu.async_copy(acc_v, out_hbm.at[pl.ds(base + ci * CHUNK, CHUNK)],
                             st)

        fire(0, rows0, sem0)

        def body(i, carry):
            ci = i * 2
            fire(ci + 1, rows1, sem1)
            reduce_store(ci, rows0, sem0, acc0, st0)

            @pl.when(ci + 2 < n_chunks)
            def _():
                fire(ci + 2, rows0, sem0)

            reduce_store(ci + 1, rows1, sem1, acc1, st1)
            return carry

        lax.fori_loop(0, n_chunks // 2, body, 0)
        pltpu.make_async_copy(acc0, out_hbm.at[pl.ds(base, CHUNK)], st0).wait()
        pltpu.make_async_copy(acc1, out_hbm.at[pl.ds(base, CHUNK)], st1).wait()

    return gsum


def _gather_bonds(table, idx):
    return _make_gather_sum(50176)(table, idx)


def _gather_atoms(table, idx):
    return _make_gather_sum(25088)(table, idx)


def _prep_idx(g, n_pad):
    n = g.shape[0]
    pad = (np.arange((n_pad - n) * MAX_NB, dtype=np.int32) * 997) % N_TOT
    g = jnp.concatenate(
        [g, jnp.asarray(pad.reshape(n_pad - n, MAX_NB))], axis=0)
    return g.reshape(NW, n_pad // NW // CHUNK, CHUNK * MAX_NB)


_MM_DIMS = (((1,), (1,)), ((), ()))


def _tc1_body(fb_ref, wi_ref, tree_ref, bin_ref, msg_ref):
    i = pl.program_id(0)
    b = lax.dot_general(fb_ref[...], wi_ref[...], _MM_DIMS,
                        preferred_element_type=jnp.float32)
    bin_ref[...] = b
    msg_ref[...] = jnp.where(i < 15, tree_ref[...], jnp.maximum(b, 0.0))


def _tc2_body(tree_ref, bin_ref, nei_ref, wh_ref, msg_ref):
    i = pl.program_id(0)
    nm = lax.dot_general(nei_ref[...], wh_ref[...], _MM_DIMS,
                         preferred_element_type=jnp.float32)
    msg_ref[...] = jnp.where(i < 15, tree_ref[...],
                             jnp.maximum(bin_ref[...] + nm, 0.0))


def _tc3_body(fat_ref, nei_ref, wo1_ref, wo2_ref, par_ref, hid_ref, st_ref):
    i = pl.program_id(0)
    a = lax.dot_general(fat_ref[...], wo1_ref[...], _MM_DIMS,
                        preferred_element_type=jnp.float32)
    a = a + lax.dot_general(nei_ref[...], wo2_ref[...], _MM_DIMS,
                            preferred_element_type=jnp.float32)
    a = a + par_ref[0:1, :]
    hid_ref[...] = a

    @pl.when(i == 0)
    def _():
        st_ref[...] = jnp.zeros_like(st_ref)

    st_ref[0:1, :] += jnp.sum(a, axis=0, keepdims=True)
    st_ref[1:2, :] += jnp.sum(a * a, axis=0, keepdims=True)


def _tc4_body(hid_ref, st_ref, par_ref, mol_ref):
    inv_n = 1.0 / N_ATOMS
    mean = st_ref[0:1, :] * inv_n
    var = st_ref[1:2, :] * inv_n - mean * mean
    inv = lax.rsqrt(var + 1e-5)
    scale = inv * par_ref[1:2, :]
    shift = par_ref[2:3, :] - mean * scale
    x = hid_ref[...]
    y = jnp.maximum(x * scale[:, None, :] + shift[:, None, :], 0.0)
    mol_ref[...] = jnp.sum(y, axis=1) * (1.0 / APM)


def kernel(fatoms, fbonds, agraph, bgraph, scope, tree_message, W_i, W_h,
           W_o, b_o, bn_gamma, bn_beta):
    del scope
    bg = _prep_idx(bgraph, 50176)
    ag = _prep_idx(agraph, 25088)
    params = (jnp.zeros((8, H), jnp.float32)
              .at[0].set(b_o).at[1].set(bn_gamma).at[2].set(bn_beta))
    wo1 = W_o[:, :AF]
    wo2 = W_o[:, AF:]

    bspec = pl.BlockSpec((1000, H), lambda i: (jnp.maximum(i - 15, 0), 0))
    tspec = pl.BlockSpec((1000, H), lambda i: (jnp.minimum(i, 14), 0))

    binput, msg = pl.pallas_call(
        _tc1_body,
        grid=(65,),
        in_specs=[
            pl.BlockSpec((1000, BF), lambda i: (jnp.maximum(i - 15, 0), 0)),
            pl.BlockSpec((H, BF), lambda i: (0, 0)),
            tspec,
        ],
        out_specs=[bspec, pl.BlockSpec((1000, H), lambda i: (i, 0))],
        out_shape=[jax.ShapeDtypeStruct((N_BONDS, H), jnp.float32),
                   jax.ShapeDtypeStruct((N_TOT, H), jnp.float32)],
    )(fbonds, W_i, tree_message)

    for _ in range(2):
        nei = _gather_bonds(msg, bg)[:N_BONDS]
        msg = pl.pallas_call(
            _tc2_body,
            grid=(65,),
            in_specs=[tspec, bspec, bspec,
                      pl.BlockSpec((H, H), lambda i: (0, 0))],
            out_specs=pl.BlockSpec((1000, H), lambda i: (i, 0)),
            out_shape=jax.ShapeDtypeStruct((N_TOT, H), jnp.float32),
        )(tree_message, binput, nei, W_h)

    nei_a = _gather_atoms(msg, ag)[:N_ATOMS]

    hid, stats = pl.pallas_call(
        _tc3_body,
        grid=(25,),
        in_specs=[
            pl.BlockSpec((1000, AF), lambda i: (i, 0)),
            pl.BlockSpec((1000, H), lambda i: (i, 0)),
            pl.BlockSpec((H, AF), lambda i: (0, 0)),
            pl.BlockSpec((H, H), lambda i: (0, 0)),
            pl.BlockSpec((8, H), lambda i: (0, 0)),
        ],
        out_specs=[pl.BlockSpec((1000, H), lambda i: (i, 0)),
                   pl.BlockSpec((8, H), lambda i: (0, 0))],
        out_shape=[jax.ShapeDtypeStruct((N_ATOMS, H), jnp.float32),
                   jax.ShapeDtypeStruct((8, H), jnp.float32)],
    )(fatoms, nei_a, wo1, wo2, params)

    mol_vecs = pl.pallas_call(
        _tc4_body,
        grid=(1,),
        in_specs=[
            pl.BlockSpec((N_MOLS, APM, H), lambda i: (0, 0, 0)),
            pl.BlockSpec((8, H), lambda i: (0, 0)),
            pl.BlockSpec((8, H), lambda i: (0, 0)),
        ],
        out_specs=pl.BlockSpec((N_MOLS, H), lambda i: (0, 0)),
        out_shape=jax.ShapeDtypeStruct((N_MOLS, H), jnp.float32),
    )(hid.reshape(N_MOLS, APM, H), stats, params)

    return mol_vecs


# speedup vs baseline: 1.7061x; 1.3534x over previous
import functools

import jax
import jax.numpy as jnp
import numpy as np
from jax import lax
from jax.experimental import pallas as pl
from jax.experimental.pallas import tpu as pltpu
from jax.experimental.pallas import tpu_sc as plsc

H = 128
MAX_NB = 15
N_ATOMS = 25000
N_BONDS = 50000
N_MESS = 15000
N_TOT = N_MESS + N_BONDS
N_MOLS = 625
APM = 40
AF = 35
BF = 40

NC, NS = 2, 16
NW = NC * NS
CHUNK = 8


NBUF = 2
SLAB = CHUNK * MAX_NB

@functools.cache
def _make_gather_sum(n_rows_pad, nbuf):
    per_w = n_rows_pad // NW
    n_chunks = per_w // CHUNK
    assert per_w * NW == n_rows_pad and n_chunks * CHUNK == per_w
    assert n_chunks % nbuf == 0
    mesh = plsc.VectorSubcoreMesh(core_axis_name="c", subcore_axis_name="s",
                                  num_cores=NC, num_subcores=NS)

    @functools.partial(
        pl.kernel,
        mesh=mesh,
        out_type=jax.ShapeDtypeStruct((n_rows_pad, H), jnp.float32),
        scratch_types=(
            [pltpu.VMEM((n_chunks, SLAB), jnp.int32)]
            + [pltpu.VMEM((SLAB, H), jnp.float32)] * nbuf
            + [pltpu.VMEM((CHUNK, H), jnp.float32)] * nbuf
            + [pltpu.SemaphoreType.DMA] * (2 * nbuf)
        ),
    )
    def gsum(table_hbm, idx_hbm, out_hbm, idx_all, *bufs):
        rows = bufs[:nbuf]
        accs = bufs[nbuf:2 * nbuf]
        sems = bufs[2 * nbuf:3 * nbuf]
        sts = bufs[3 * nbuf:4 * nbuf]
        wid = lax.axis_index("s") * NC + lax.axis_index("c")
        base = wid * per_w
        pltpu.sync_copy(idx_hbm.at[wid], idx_all)

        def fire(c, k):
            pltpu.async_copy(table_hbm.at[idx_all.at[c]], rows[k], sems[k])

        for k in range(nbuf):
            fire(k, k)

        def body(i, carry):
            for k in range(nbuf):
                c = i * nbuf + k
                pltpu.make_async_copy(table_hbm.at[idx_all.at[c]], rows[k],
                                      sems[k]).wait()

                @pl.when(c >= nbuf)
                def _():
                    pltpu.make_async_copy(
                        accs[k], out_hbm.at[pl.ds(base, CHUNK)], sts[k]).wait()

                def bond(b, carry2, k=k):
                    r = b * MAX_NB
                    for v in range(H // 16):
                        sl = pl.ds(v * 16, 16)
                        acc = rows[k][r, sl]
                        for j in range(1, MAX_NB):
                            acc = acc + rows[k][r + j, sl]
                        accs[k][b, sl] = acc
                    return carry2

                lax.fori_loop(0, CHUNK, bond, 0)
                pltpu.async_copy(accs[k],
                                 out_hbm.at[pl.ds(base + c * CHUNK, CHUNK)],
                                 sts[k])

                @pl.when(c + nbuf < n_chunks)
                def _():
                    fire(c + nbuf, k)
            return carry

        lax.fori_loop(0, n_chunks // nbuf, body, 0)
        for k in range(nbuf):
            pltpu.make_async_copy(accs[k], out_hbm.at[pl.ds(base, CHUNK)],
                                  sts[k]).wait()

    return gsum


def _gather_bonds(table, idx):
    return _make_gather_sum(50176, 4)(table, idx)


def _gather_atoms(table, idx):
    return _make_gather_sum(25088, 2)(table, idx)


def _prep_idx(g, n_pad):
    n = g.shape[0]
    pad = (np.arange((n_pad - n) * MAX_NB, dtype=np.int32) * 997) % N_TOT
    g = jnp.concatenate(
        [g, jnp.asarray(pad.reshape(n_pad - n, MAX_NB))], axis=0)
    return g.reshape(NW, n_pad // NW // CHUNK, CHUNK * MAX_NB)


_MM_DIMS = (((1,), (1,)), ((), ()))


def _tc1_body(fb_ref, wi_ref, tree_ref, bin_ref, msg_ref):
    i = pl.program_id(0)
    b = lax.dot_general(fb_ref[...], wi_ref[...], _MM_DIMS,
                        preferred_element_type=jnp.float32)
    bin_ref[...] = b
    msg_ref[...] = jnp.where(i < 15, tree_ref[...], jnp.maximum(b, 0.0))


def _tc2_body(tree_ref, bin_ref, nei_ref, wh_ref, msg_ref):
    i = pl.program_id(0)
    nm = lax.dot_general(nei_ref[...], wh_ref[...], _MM_DIMS,
                         preferred_element_type=jnp.float32)
    msg_ref[...] = jnp.where(i < 15, tree_ref[...],
                             jnp.maximum(bin_ref[...] + nm, 0.0))


def _tc3_body(fat_ref, nei_ref, wo1_ref, wo2_ref, par_ref, hid_ref, st_ref):
    i = pl.program_id(0)
    a = lax.dot_general(fat_ref[...], wo1_ref[...], _MM_DIMS,
                        preferred_element_type=jnp.float32)
    a = a + lax.dot_general(nei_ref[...], wo2_ref[...], _MM_DIMS,
                            preferred_element_type=jnp.float32)
    a = a + par_ref[0:1, :]
    hid_ref[...] = a

    @pl.when(i == 0)
    def _():
        st_ref[...] = jnp.zeros_like(st_ref)

    st_ref[0:1, :] += jnp.sum(a, axis=0, keepdims=True)
    st_ref[1:2, :] += jnp.sum(a * a, axis=0, keepdims=True)


def _tc4_body(hid_ref, st_ref, par_ref, mol_ref):
    inv_n = 1.0 / N_ATOMS
    mean = st_ref[0:1, :] * inv_n
    var = st_ref[1:2, :] * inv_n - mean * mean
    inv = lax.rsqrt(var + 1e-5)
    scale = inv * par_ref[1:2, :]
    shift = par_ref[2:3, :] - mean * scale
    x = hid_ref[...]
    y = jnp.maximum(x * scale[:, None, :] + shift[:, None, :], 0.0)
    mol_ref[...] = jnp.sum(y, axis=1) * (1.0 / APM)


def kernel(fatoms, fbonds, agraph, bgraph, scope, tree_message, W_i, W_h,
           W_o, b_o, bn_gamma, bn_beta):
    del scope
    bg = _prep_idx(bgraph, 50176)
    ag = _prep_idx(agraph, 25088)
    params = (jnp.zeros((8, H), jnp.float32)
              .at[0].set(b_o).at[1].set(bn_gamma).at[2].set(bn_beta))
    wo1 = W_o[:, :AF]
    wo2 = W_o[:, AF:]

    bspec = pl.BlockSpec((1000, H), lambda i: (jnp.maximum(i - 15, 0), 0))
    tspec = pl.BlockSpec((1000, H), lambda i: (jnp.minimum(i, 14), 0))

    binput, msg = pl.pallas_call(
        _tc1_body,
        grid=(65,),
        in_specs=[
            pl.BlockSpec((1000, BF), lambda i: (jnp.maximum(i - 15, 0), 0)),
            pl.BlockSpec((H, BF), lambda i: (0, 0)),
            tspec,
        ],
        out_specs=[bspec, pl.BlockSpec((1000, H), lambda i: (i, 0))],
        out_shape=[jax.ShapeDtypeStruct((N_BONDS, H), jnp.float32),
                   jax.ShapeDtypeStruct((N_TOT, H), jnp.float32)],
    )(fbonds, W_i, tree_message)

    for _ in range(2):
        nei = _gather_bonds(msg, bg)[:N_BONDS]
        msg = pl.pallas_call(
            _tc2_body,
            grid=(65,),
            in_specs=[tspec, bspec, bspec,
                      pl.BlockSpec((H, H), lambda i: (0, 0))],
            out_specs=pl.BlockSpec((1000, H), lambda i: (i, 0)),
            out_shape=jax.ShapeDtypeStruct((N_TOT, H), jnp.float32),
        )(tree_message, binput, nei, W_h)

    nei_a = _gather_atoms(msg, ag)[:N_ATOMS]

    hid, stats = pl.pallas_call(
        _tc3_body,
        grid=(25,),
        in_specs=[
            pl.BlockSpec((1000, AF), lambda i: (i, 0)),
            pl.BlockSpec((1000, H), lambda i: (i, 0)),
            pl.BlockSpec((H, AF), lambda i: (0, 0)),
            pl.BlockSpec((H, H), lambda i: (0, 0)),
            pl.BlockSpec((8, H), lambda i: (0, 0)),
        ],
        out_specs=[pl.BlockSpec((1000, H), lambda i: (i, 0)),
                   pl.BlockSpec((8, H), lambda i: (0, 0))],
        out_shape=[jax.ShapeDtypeStruct((N_ATOMS, H), jnp.float32),
                   jax.ShapeDtypeStruct((8, H), jnp.float32)],
    )(fatoms, nei_a, wo1, wo2, params)

    mol_vecs = pl.pallas_call(
        _tc4_body,
        grid=(1,),
        in_specs=[
            pl.BlockSpec((N_MOLS, APM, H), lambda i: (0, 0, 0)),
            pl.BlockSpec((8, H), lambda i: (0, 0)),
            pl.BlockSpec((8, H), lambda i: (0, 0)),
        ],
        out_specs=pl.BlockSpec((N_MOLS, H), lambda i: (0, 0)),
        out_shape=jax.ShapeDtypeStruct((N_MOLS, H), jnp.float32),
    )(hid.reshape(N_MOLS, APM, H), stats, params)

    return mol_vecs
